# Initial kernel scaffold; baseline (speedup 1.0000x reference)
import jax
import jax.numpy as jnp
from jax.experimental import pallas as pl


def kernel(users, items, adj_row, adj_col, adj_val, user_emb, item_emb):
    raise NotImplementedError("write your pallas kernel here")


# speedup vs baseline: 5.8756x; 5.8756x over previous
import functools

import jax
import jax.numpy as jnp
from jax import lax
from jax.experimental import pallas as pl
from jax.experimental.pallas import tpu as pltpu
from jax.experimental.pallas import tpu_sc as plsc

NC = 2
NS = 16
L = 16
NW = NC * NS
DIM = 64
ND = DIM // L
ROWS_W = 1568
N_PAD = ROWS_W * NW
C = 128
UNROLL = 8


def _worker_id():
    return lax.axis_index("s") * NC + lax.axis_index("c")


def _make_layer(e_pad):
    mesh = plsc.VectorSubcoreMesh(core_axis_name="c", subcore_axis_name="s")

    @functools.partial(
        pl.kernel,
        mesh=mesh,
        compiler_params=pltpu.CompilerParams(needs_layout_passes=False, use_tc_tiling_on_sc=False),
        out_type=jax.ShapeDtypeStruct((N_PAD, DIM), jnp.float32),
        scratch_types=[
            pltpu.VMEM((64,), jnp.int32),
            pltpu.VMEM((C,), jnp.int32),
            pltpu.VMEM((C,), jnp.int32),
            pltpu.VMEM((C,), jnp.float32),
            pltpu.VMEM((C,), jnp.int32),
            pltpu.VMEM((C,), jnp.float32),
            pltpu.VMEM((C, DIM), jnp.float32),
            pltpu.VMEM((ROWS_W, DIM), jnp.float32),
            pltpu.SemaphoreType.DMA,
        ],
    )
    def layer(ego_hbm, row_hbm, col_hbm, val_hbm, starts_hbm, out_hbm,
              starts_v, colidx, rowch, valch, rowloc, valeff, gath, acc, sem):
        wid = _worker_id()
        r0 = wid * ROWS_W

        pltpu.sync_copy(starts_hbm, starts_v)
        widv = jnp.full((L,), wid, jnp.int32)
        start = jnp.max(plsc.load_gather(starts_v, [widv]))
        end = jnp.max(plsc.load_gather(starts_v, [widv + 1]))
        start_al = jnp.bitwise_and(start, jnp.int32(-8))
        nchunks = (end - start_al + (C - 1)) // C

        zero = jnp.zeros((L,), jnp.float32)

        def _zero_rows(r, carry):
            for d in range(ND):
                acc[r, pl.ds(d * L, L)] = zero
            return carry

        lax.fori_loop(0, ROWS_W, _zero_rows, 0)

        dim_iota = [d * L + lax.broadcasted_iota(jnp.int32, (L,), 0)
                    for d in range(ND)]

        def _chunk(k, carry):
            e = pl.multiple_of(start_al + k * C, 8)
            pltpu.sync_copy(col_hbm.at[pl.ds(e, C)], colidx)
            pltpu.sync_copy(row_hbm.at[pl.ds(e, C)], rowch)
            pltpu.sync_copy(val_hbm.at[pl.ds(e, C)], valch)
            pltpu.async_copy(ego_hbm.at[colidx], gath, sem).wait()

            for i in range(C // L):
                sl = pl.ds(i * L, L)
                rl = rowch[sl] - r0
                ok = (rl >= 0) & (rl < ROWS_W)
                rowloc[sl] = jnp.clip(rl, 0, ROWS_W - 1)
                valeff[sl] = jnp.where(ok, valch[sl], jnp.float32(0.0))

            def _edges(g, carry2):
                for u in range(UNROLL):
                    c = g * UNROLL + u
                    cv = jnp.full((L,), c, jnp.int32)
                    vv = plsc.load_gather(valeff, [cv])
                    rv = plsc.load_gather(rowloc, [cv])
                    for d in range(ND):
                        gvec = gath[c, pl.ds(d * L, L)]
                        plsc.addupdate_scatter(acc, [rv, dim_iota[d]],
                                               gvec * vv)
                return carry2

            lax.fori_loop(0, C // UNROLL, _edges, 0)
            return carry

        lax.fori_loop(0, nchunks, _chunk, 0)
        pltpu.sync_copy(acc, out_hbm.at[pl.ds(r0, ROWS_W)])

    return layer


def _make_final(batch):
    bpw = batch // NW
    mesh = plsc.VectorSubcoreMesh(core_axis_name="c", subcore_axis_name="s")
    out_sds = jax.ShapeDtypeStruct((batch, DIM), jnp.float32)

    @functools.partial(
        pl.kernel,
        mesh=mesh,
        compiler_params=pltpu.CompilerParams(needs_layout_passes=False, use_tc_tiling_on_sc=False),
        out_type=(out_sds, out_sds),
        scratch_types=[
            pltpu.VMEM((bpw,), jnp.int32),
            pltpu.VMEM((bpw, DIM), jnp.float32),
            pltpu.VMEM((bpw, DIM), jnp.float32),
            pltpu.VMEM((bpw, DIM), jnp.float32),
            pltpu.VMEM((bpw, DIM), jnp.float32),
            pltpu.VMEM((bpw, DIM), jnp.float32),
            pltpu.SemaphoreType.DMA,
        ],
    )
    def final(e0, e1, e2, e3, uidx_hbm, iidx_hbm, uout_hbm, iout_hbm,
              idxv, g0, g1, g2, g3, obuf, sem):
        wid = _worker_id()
        b0 = wid * bpw
        quarter = jnp.float32(0.25)
        for idx_hbm, out_hbm in ((uidx_hbm, uout_hbm), (iidx_hbm, iout_hbm)):
            pltpu.sync_copy(idx_hbm.at[pl.ds(b0, bpw)], idxv)
            for tab, gb in ((e0, g0), (e1, g1), (e2, g2), (e3, g3)):
                pltpu.async_copy(tab.at[idxv], gb, sem).wait()

            def _avg(r, carry):
                for d in range(ND):
                    sl = pl.ds(d * L, L)
                    obuf[r, sl] = (g0[r, sl] + g1[r, sl] + g2[r, sl]
                                   + g3[r, sl]) * quarter
                return carry

            lax.fori_loop(0, bpw, _avg, 0)
            pltpu.sync_copy(obuf, out_hbm.at[pl.ds(b0, bpw)])

    return final


def kernel(users, items, adj_row, adj_col, adj_val, user_emb, item_emb):
    nu, dim = user_emb.shape
    ni = item_emb.shape[0]
    n = nu + ni
    assert dim == DIM and n <= N_PAD
    e_cnt = adj_row.shape[0]

    mkey = jax.random.key(42)
    random_tensor = 0.5 + jax.random.uniform(mkey, adj_val.shape)
    mask = jnp.floor(random_tensor).astype(bool)
    vals = jnp.where(mask, adj_val, 0.0) * 2.0

    adj_row = adj_row.astype(jnp.int32)
    adj_col = adj_col.astype(jnp.int32)

    e_pad = ((e_cnt + C - 1) // C + 1) * C
    pe = e_pad - e_cnt
    rows_p = jnp.pad(adj_row, (0, pe))
    cols_p = jnp.pad(adj_col, (0, pe))
    vals_p = jnp.pad(vals, (0, pe))

    bounds = jnp.arange(NW + 1, dtype=jnp.int32) * ROWS_W
    starts = jnp.searchsorted(adj_row, bounds, side="left").astype(jnp.int32)
    starts64 = jnp.zeros((64,), jnp.int32).at[: NW + 1].set(starts)

    ego0 = jnp.pad(jnp.concatenate([user_emb, item_emb], axis=0),
                   ((0, N_PAD - n), (0, 0)))

    layer = _make_layer(e_pad)
    e1 = layer(ego0, rows_p, cols_p, vals_p, starts64)
    e2 = layer(e1, rows_p, cols_p, vals_p, starts64)
    e3 = layer(e2, rows_p, cols_p, vals_p, starts64)

    batch = users.shape[0]
    uidx = users.astype(jnp.int32)
    iidx = items.astype(jnp.int32) + nu
    final = _make_final(batch)
    u_out, i_out = final(ego0, e1, e2, e3, uidx, iidx)
    return (u_out, i_out)


# speedup vs baseline: 10.6757x; 1.8169x over previous
import functools

import jax
import jax.numpy as jnp
from jax import lax
from jax.experimental import pallas as pl
from jax.experimental.pallas import tpu as pltpu
from jax.experimental.pallas import tpu_sc as plsc

NC = 2
NS = 16
L = 16
NW = NC * NS
DIM = 64
ND = DIM // L
ROWS_W = 1568
N_PAD = ROWS_W * NW
C = 128
UNROLL = 8

_params = pltpu.CompilerParams(needs_layout_passes=False,
                               use_tc_tiling_on_sc=False)


def _worker_id():
    return lax.axis_index("s") * NC + lax.axis_index("c")


def _make_layer(e_pad):
    mesh = plsc.VectorSubcoreMesh(core_axis_name="c", subcore_axis_name="s")

    @functools.partial(
        pl.kernel,
        mesh=mesh,
        compiler_params=_params,
        out_type=jax.ShapeDtypeStruct((N_PAD * DIM,), jnp.float32),
        scratch_types=[
            pltpu.VMEM((64,), jnp.int32),
            pltpu.VMEM((C,), jnp.int32),
            pltpu.VMEM((C,), jnp.int32),
            pltpu.VMEM((C,), jnp.int32),
            pltpu.VMEM((C,), jnp.int32),
            pltpu.VMEM((C,), jnp.float32),
            pltpu.VMEM((C,), jnp.float32),
            pltpu.VMEM((C,), jnp.int32),
            pltpu.VMEM((C,), jnp.float32),
            pltpu.VMEM((C, DIM), jnp.float32),
            pltpu.VMEM((C, DIM), jnp.float32),
            pltpu.VMEM((ROWS_W * DIM,), jnp.float32),
            pltpu.SemaphoreType.DMA,
            pltpu.SemaphoreType.DMA,
            pltpu.SemaphoreType.DMA,
            pltpu.SemaphoreType.DMA,
        ],
    )
    def layer(ego_hbm, row_hbm, col_hbm, val_hbm, starts_hbm, out_hbm,
              starts_v, col_a, col_b, row_a, row_b, val_a, val_b,
              base, veff, gath_a, gath_b, acc,
              esem_a, esem_b, gsem_a, gsem_b):
        wid = _worker_id()
        r0 = wid * ROWS_W

        pltpu.sync_copy(starts_hbm, starts_v)
        widv = jnp.full((L,), wid, jnp.int32)
        start = jnp.max(plsc.load_gather(starts_v, [widv]))
        end = jnp.max(plsc.load_gather(starts_v, [widv + 1]))
        start_al = jnp.bitwise_and(start, jnp.int32(-8))
        nchunks = (end - start_al + (C - 1)) // C
        nchunks = jnp.maximum(nchunks, 1)
        npairs = (nchunks + 1) // 2

        bufs = ((col_a, row_a, val_a, gath_a, esem_a, gsem_a),
                (col_b, row_b, val_b, gath_b, esem_b, gsem_b))

        def _issue_edges(b, e):
            col, row, val = bufs[b][0], bufs[b][1], bufs[b][2]
            esem = bufs[b][4]
            pltpu.async_copy(col_hbm.at[pl.ds(e, C)], col, esem)
            pltpu.async_copy(row_hbm.at[pl.ds(e, C)], row, esem)
            pltpu.async_copy(val_hbm.at[pl.ds(e, C)], val, esem)

        def _wait_edges(b, e):
            col, row, val = bufs[b][0], bufs[b][1], bufs[b][2]
            esem = bufs[b][4]
            pltpu.make_async_copy(col_hbm.at[pl.ds(e, C)], col, esem).wait()
            pltpu.make_async_copy(row_hbm.at[pl.ds(e, C)], row, esem).wait()
            pltpu.make_async_copy(val_hbm.at[pl.ds(e, C)], val, esem).wait()

        def _issue_gather(b):
            col, gath, gsem = bufs[b][0], bufs[b][3], bufs[b][5]
            pltpu.async_copy(ego_hbm.at[col], gath, gsem)

        def _wait_gather(b):
            col, gath, gsem = bufs[b][0], bufs[b][3], bufs[b][5]
            pltpu.make_async_copy(ego_hbm.at[col], gath, gsem).wait()

        e0 = pl.multiple_of(start_al, 8)
        _issue_edges(0, e0)

        zero = jnp.zeros((L,), jnp.float32)

        def _zero_rows(r, carry):
            for u in range(UNROLL):
                acc[pl.ds((r * UNROLL + u) * L, L)] = zero
            return carry

        lax.fori_loop(0, ROWS_W * DIM // (L * UNROLL), _zero_rows, 0)

        _wait_edges(0, e0)
        _issue_gather(0)
        _issue_edges(1, pl.multiple_of(start_al + C, 8))

        dim_iota = [d * L + lax.broadcasted_iota(jnp.int32, (L,), 0)
                    for d in range(ND)]

        def _pair(kk, carry):
            for b in (0, 1):
                k = kk * 2 + b
                row, gath = bufs[b][1], bufs[b][3]
                val = bufs[b][2]
                nb = 1 - b
                e_next = pl.multiple_of(start_al + (k + 1) * C, 8)
                e_next2 = pl.multiple_of(start_al + (k + 2) * C, 8)

                _wait_gather(b)

                for i in range(C // L):
                    sl = pl.ds(i * L, L)
                    rl = row[sl] - r0
                    ok = (rl >= 0) & (rl < ROWS_W)
                    base[sl] = jnp.clip(rl, 0, ROWS_W - 1) * DIM
                    veff[sl] = jnp.where(ok, val[sl], jnp.float32(0.0))

                _wait_edges(nb, e_next)
                _issue_gather(nb)
                _issue_edges(b, e_next2)

                def _edges(g, carry2):
                    for u in range(UNROLL):
                        c = g * UNROLL + u
                        cv = jnp.full((L,), c, jnp.int32)
                        vv = plsc.load_gather(veff, [cv])
                        bv = plsc.load_gather(base, [cv])
                        for d in range(ND):
                            gvec = gath[c, pl.ds(d * L, L)]
                            plsc.addupdate_scatter(acc, [bv + dim_iota[d]],
                                                   gvec * vv)
                    return carry2

                lax.fori_loop(0, C // UNROLL, _edges, 0)
            return carry

        lax.fori_loop(0, npairs, _pair, 0)

        last = 2 * npairs
        _wait_gather(0)
        _wait_edges(1, pl.multiple_of(start_al + (last + 1) * C, 8))

        pltpu.sync_copy(acc, out_hbm.at[pl.ds(r0 * DIM, ROWS_W * DIM)])

    return layer


def _make_final(batch):
    bpw = batch // NW
    mesh = plsc.VectorSubcoreMesh(core_axis_name="c", subcore_axis_name="s")
    out_sds = jax.ShapeDtypeStruct((batch, DIM), jnp.float32)

    @functools.partial(
        pl.kernel,
        mesh=mesh,
        compiler_params=_params,
        out_type=(out_sds, out_sds),
        scratch_types=[
            pltpu.VMEM((bpw,), jnp.int32),
            pltpu.VMEM((bpw, DIM), jnp.float32),
            pltpu.VMEM((bpw, DIM), jnp.float32),
            pltpu.VMEM((bpw, DIM), jnp.float32),
            pltpu.VMEM((bpw, DIM), jnp.float32),
            pltpu.VMEM((bpw, DIM), jnp.float32),
            pltpu.SemaphoreType.DMA,
        ],
    )
    def final(e0, e1, e2, e3, uidx_hbm, iidx_hbm, uout_hbm, iout_hbm,
              idxv, g0, g1, g2, g3, obuf, sem):
        wid = _worker_id()
        b0 = wid * bpw
        quarter = jnp.float32(0.25)
        for idx_hbm, out_hbm in ((uidx_hbm, uout_hbm), (iidx_hbm, iout_hbm)):
            pltpu.sync_copy(idx_hbm.at[pl.ds(b0, bpw)], idxv)
            for tab, gb in ((e0, g0), (e1, g1), (e2, g2), (e3, g3)):
                pltpu.async_copy(tab.at[idxv], gb, sem).wait()

            def _avg(r, carry):
                for d in range(ND):
                    sl = pl.ds(d * L, L)
                    obuf[r, sl] = (g0[r, sl] + g1[r, sl] + g2[r, sl]
                                   + g3[r, sl]) * quarter
                return carry

            lax.fori_loop(0, bpw, _avg, 0)
            pltpu.sync_copy(obuf, out_hbm.at[pl.ds(b0, bpw)])

    return final


def kernel(users, items, adj_row, adj_col, adj_val, user_emb, item_emb):
    nu, dim = user_emb.shape
    ni = item_emb.shape[0]
    n = nu + ni
    assert dim == DIM and n <= N_PAD
    e_cnt = adj_row.shape[0]

    mkey = jax.random.key(42)
    random_tensor = 0.5 + jax.random.uniform(mkey, adj_val.shape)
    mask = jnp.floor(random_tensor).astype(bool)
    vals = jnp.where(mask, adj_val, 0.0) * 2.0

    adj_row = adj_row.astype(jnp.int32)
    adj_col = adj_col.astype(jnp.int32)

    e_pad = ((e_cnt + C - 1) // C + 4) * C
    pe = e_pad - e_cnt
    rows_p = jnp.pad(adj_row, (0, pe))
    cols_p = jnp.pad(adj_col, (0, pe))
    vals_p = jnp.pad(vals, (0, pe))

    bounds = jnp.arange(NW + 1, dtype=jnp.int32) * ROWS_W
    starts = jnp.searchsorted(adj_row, bounds, side="left").astype(jnp.int32)
    starts64 = jnp.zeros((64,), jnp.int32).at[: NW + 1].set(starts)

    ego0 = jnp.pad(jnp.concatenate([user_emb, item_emb], axis=0),
                   ((0, N_PAD - n), (0, 0)))

    layer = _make_layer(e_pad)
    e1 = layer(ego0, rows_p, cols_p, vals_p, starts64).reshape(N_PAD, DIM)
    e2 = layer(e1, rows_p, cols_p, vals_p, starts64).reshape(N_PAD, DIM)
    e3 = layer(e2, rows_p, cols_p, vals_p, starts64).reshape(N_PAD, DIM)

    batch = users.shape[0]
    uidx = users.astype(jnp.int32)
    iidx = items.astype(jnp.int32) + nu
    final = _make_final(batch)
    u_out, i_out = final(ego0, e1, e2, e3, uidx, iidx)
    return (u_out, i_out)
